# initial kernel scaffold (unmeasured)
import jax
import jax.numpy as jnp
from jax import lax
from jax.experimental import pallas as pl
from jax.experimental.pallas import tpu as pltpu


def kernel(
    x,
):
    def body(*refs):
        pass

    out_shape = jax.ShapeDtypeStruct(..., jnp.float32)
    return pl.pallas_call(body, out_shape=out_shape)(...)


# baseline (device time: 19377 ns/iter reference)
import jax
import jax.numpy as jnp
from jax import lax
from jax.experimental import pallas as pl
from jax.experimental.pallas import tpu as pltpu

N_Y = 4
N_STEPS = N_Y - 1


def kernel(x):
    x2 = x.reshape(x.shape[1], x.shape[2])
    m, n_total = x2.shape
    n_chunk = n_total // N_Y

    def body(x_ref, out_ref, send_buf, recv_buf, send_sems, recv_sems):
        my_x = lax.axis_index("x")
        my_y = lax.axis_index("y")
        my_z = lax.axis_index("z")
        right = lax.rem(my_y + 1, N_Y)
        left = lax.rem(my_y + N_Y - 1, N_Y)

        barrier_sem = pltpu.get_barrier_semaphore()
        for nbr in (left, right):
            pl.semaphore_signal(
                barrier_sem, inc=1,
                device_id=(my_x, nbr, my_z),
                device_id_type=pl.DeviceIdType.MESH,
            )
        pl.semaphore_wait(barrier_sem, 2)

        def chunk(c):
            return x_ref[:, pl.ds(c * n_chunk, n_chunk)]

        for s in range(N_STEPS):
            c_send = lax.rem(my_y + 2 * N_Y - 1 - s, N_Y)
            if s == 0:
                send_buf[s, :, :] = chunk(c_send)
            else:
                send_buf[s, :, :] = recv_buf[s - 1, :, :] + chunk(c_send)
            rdma = pltpu.make_async_remote_copy(
                src_ref=send_buf.at[s],
                dst_ref=recv_buf.at[s],
                send_sem=send_sems.at[s],
                recv_sem=recv_sems.at[s],
                device_id=(my_x, right, my_z),
                device_id_type=pl.DeviceIdType.MESH,
            )
            rdma.start()
            rdma.wait()

        out_ref[...] = recv_buf[N_STEPS - 1, :, :] + chunk(my_y)

    return pl.pallas_call(
        body,
        out_shape=jax.ShapeDtypeStruct((m, n_chunk), x2.dtype),
        in_specs=[pl.BlockSpec(memory_space=pltpu.VMEM)],
        out_specs=pl.BlockSpec(memory_space=pltpu.VMEM),
        scratch_shapes=[
            pltpu.VMEM((N_STEPS, m, n_chunk), x2.dtype),
            pltpu.VMEM((N_STEPS, m, n_chunk), x2.dtype),
            pltpu.SemaphoreType.DMA((N_STEPS,)),
            pltpu.SemaphoreType.DMA((N_STEPS,)),
        ],
        compiler_params=pltpu.CompilerParams(collective_id=0),
    )(x2)


# device time: 16785 ns/iter; 1.1544x vs baseline; 1.1544x over previous
import jax
import jax.numpy as jnp
from jax import lax
from jax.experimental import pallas as pl
from jax.experimental.pallas import tpu as pltpu

N_Y = 4


def kernel(x):
    x2 = x.reshape(x.shape[1], x.shape[2])
    m, n_total = x2.shape
    n_chunk = n_total // N_Y

    def body(x_ref, out_ref, recv_buf, send_sems, recv_sems):
        my_x = lax.axis_index("x")
        my_y = lax.axis_index("y")
        my_z = lax.axis_index("z")

        barrier_sem = pltpu.get_barrier_semaphore()
        for k in range(1, N_Y):
            peer = lax.rem(my_y + k, N_Y)
            pl.semaphore_signal(
                barrier_sem, inc=1,
                device_id=(my_x, peer, my_z),
                device_id_type=pl.DeviceIdType.MESH,
            )
        pl.semaphore_wait(barrier_sem, N_Y - 1)

        sends = []
        for k in range(1, N_Y):
            dst = lax.rem(my_y + k, N_Y)
            rdma = pltpu.make_async_remote_copy(
                src_ref=x_ref.at[:, pl.ds(dst * n_chunk, n_chunk)],
                dst_ref=recv_buf.at[my_y],
                send_sem=send_sems.at[k - 1],
                recv_sem=recv_sems.at[my_y],
                device_id=(my_x, dst, my_z),
                device_id_type=pl.DeviceIdType.MESH,
            )
            rdma.start()
            sends.append(rdma)

        for k in range(1, N_Y):
            src = lax.rem(my_y + k, N_Y)
            recv = pltpu.make_async_remote_copy(
                src_ref=recv_buf.at[src],
                dst_ref=recv_buf.at[src],
                send_sem=send_sems.at[k - 1],
                recv_sem=recv_sems.at[src],
                device_id=(my_x, src, my_z),
                device_id_type=pl.DeviceIdType.MESH,
            )
            recv.wait_recv()

        s1 = lax.rem(my_y + 1, N_Y)
        s2 = lax.rem(my_y + 2, N_Y)
        s3 = lax.rem(my_y + 3, N_Y)
        out_ref[...] = (
            x_ref[:, pl.ds(my_y * n_chunk, n_chunk)]
            + recv_buf[s1, :, :]
            + recv_buf[s2, :, :]
            + recv_buf[s3, :, :]
        )

        for rdma in sends:
            rdma.wait_send()

    return pl.pallas_call(
        body,
        out_shape=jax.ShapeDtypeStruct((m, n_chunk), x2.dtype),
        in_specs=[pl.BlockSpec(memory_space=pltpu.VMEM)],
        out_specs=pl.BlockSpec(memory_space=pltpu.VMEM),
        scratch_shapes=[
            pltpu.VMEM((N_Y, m, n_chunk), x2.dtype),
            pltpu.SemaphoreType.DMA((N_Y - 1,)),
            pltpu.SemaphoreType.DMA((N_Y,)),
        ],
        compiler_params=pltpu.CompilerParams(collective_id=0),
    )(x2)


# device time: 15736 ns/iter; 1.2314x vs baseline; 1.0667x over previous
import jax
import jax.numpy as jnp
from jax import lax
from jax.experimental import pallas as pl
from jax.experimental.pallas import tpu as pltpu

N_Y = 4
N_X = 2


def kernel(x):
    x2 = x.reshape(x.shape[1], x.shape[2])
    m, n_total = x2.shape
    n_chunk = n_total // N_Y
    m_half = m // N_X

    def body(x_ref, out_ref, recv_buf, send_sems, recv_sems,
             send_sem_x, recv_sem_x):
        my_x = lax.axis_index("x")
        my_y = lax.axis_index("y")
        my_z = lax.axis_index("z")
        partner_x = 1 - my_x
        row0 = my_x * m_half
        prow0 = partner_x * m_half

        barrier_sem = pltpu.get_barrier_semaphore()
        for k in range(1, N_Y):
            peer = lax.rem(my_y + k, N_Y)
            pl.semaphore_signal(
                barrier_sem, inc=1,
                device_id=(my_x, peer, my_z),
                device_id_type=pl.DeviceIdType.MESH,
            )
        pl.semaphore_signal(
            barrier_sem, inc=1,
            device_id=(partner_x, my_y, my_z),
            device_id_type=pl.DeviceIdType.MESH,
        )
        pl.semaphore_wait(barrier_sem, N_Y)

        sends = []
        for k in range(1, N_Y):
            dst = lax.rem(my_y + k, N_Y)
            rdma = pltpu.make_async_remote_copy(
                src_ref=x_ref.at[pl.ds(row0, m_half),
                                 pl.ds(dst * n_chunk, n_chunk)],
                dst_ref=recv_buf.at[my_y],
                send_sem=send_sems.at[k - 1],
                recv_sem=recv_sems.at[my_y],
                device_id=(my_x, dst, my_z),
                device_id_type=pl.DeviceIdType.MESH,
            )
            rdma.start()
            sends.append(rdma)

        for k in range(1, N_Y):
            src = lax.rem(my_y + k, N_Y)
            recv = pltpu.make_async_remote_copy(
                src_ref=recv_buf.at[src],
                dst_ref=recv_buf.at[src],
                send_sem=send_sems.at[k - 1],
                recv_sem=recv_sems.at[src],
                device_id=(my_x, src, my_z),
                device_id_type=pl.DeviceIdType.MESH,
            )
            recv.wait_recv()

        s1 = lax.rem(my_y + 1, N_Y)
        s2 = lax.rem(my_y + 2, N_Y)
        s3 = lax.rem(my_y + 3, N_Y)
        out_ref[pl.ds(row0, m_half), :] = (
            x_ref[pl.ds(row0, m_half), pl.ds(my_y * n_chunk, n_chunk)]
            + recv_buf[s1, :, :]
            + recv_buf[s2, :, :]
            + recv_buf[s3, :, :]
        )

        xchg = pltpu.make_async_remote_copy(
            src_ref=out_ref.at[pl.ds(row0, m_half), :],
            dst_ref=out_ref.at[pl.ds(row0, m_half), :],
            send_sem=send_sem_x,
            recv_sem=recv_sem_x,
            device_id=(partner_x, my_y, my_z),
            device_id_type=pl.DeviceIdType.MESH,
        )
        xchg.start()
        xrecv = pltpu.make_async_remote_copy(
            src_ref=out_ref.at[pl.ds(prow0, m_half), :],
            dst_ref=out_ref.at[pl.ds(prow0, m_half), :],
            send_sem=send_sem_x,
            recv_sem=recv_sem_x,
            device_id=(partner_x, my_y, my_z),
            device_id_type=pl.DeviceIdType.MESH,
        )
        xrecv.wait_recv()
        xchg.wait_send()

        for rdma in sends:
            rdma.wait_send()

    return pl.pallas_call(
        body,
        out_shape=jax.ShapeDtypeStruct((m, n_chunk), x2.dtype),
        in_specs=[pl.BlockSpec(memory_space=pltpu.VMEM)],
        out_specs=pl.BlockSpec(memory_space=pltpu.VMEM),
        scratch_shapes=[
            pltpu.VMEM((N_Y, m_half, n_chunk), x2.dtype),
            pltpu.SemaphoreType.DMA((N_Y - 1,)),
            pltpu.SemaphoreType.DMA((N_Y,)),
            pltpu.SemaphoreType.DMA,
            pltpu.SemaphoreType.DMA,
        ],
        compiler_params=pltpu.CompilerParams(collective_id=0),
    )(x2)


# device time: 15691 ns/iter; 1.2349x vs baseline; 1.0029x over previous
import jax
from jax import lax
from jax.experimental import pallas as pl
from jax.experimental.pallas import tpu as pltpu

N_Y = 4
N_X = 2


def kernel(x):
    x2 = x.reshape(x.shape[1], x.shape[2])
    m, n_total = x2.shape
    n_chunk = n_total // N_Y
    m_half = m // N_X

    def body(x_ref, out_ref, recv_buf, send_sems, recv_sems,
             send_sem_x, recv_sem_x):
        my_x = lax.axis_index("x")
        my_y = lax.axis_index("y")
        my_z = lax.axis_index("z")
        partner_x = 1 - my_x
        row0 = my_x * m_half
        prow0 = partner_x * m_half

        barrier_sem = pltpu.get_barrier_semaphore()
        for k in range(1, N_Y):
            peer = lax.rem(my_y + k, N_Y)
            pl.semaphore_signal(
                barrier_sem, inc=1,
                device_id=(my_x, peer, my_z),
                device_id_type=pl.DeviceIdType.MESH,
            )
        pl.semaphore_signal(
            barrier_sem, inc=1,
            device_id=(partner_x, my_y, my_z),
            device_id_type=pl.DeviceIdType.MESH,
        )
        pl.semaphore_wait(barrier_sem, N_Y)

        sends = []
        for k in range(1, N_Y):
            dst = lax.rem(my_y + k, N_Y)
            slot = N_Y - 1 - k
            rdma = pltpu.make_async_remote_copy(
                src_ref=x_ref.at[pl.ds(row0, m_half),
                                 pl.ds(dst * n_chunk, n_chunk)],
                dst_ref=recv_buf.at[slot],
                send_sem=send_sems.at[k - 1],
                recv_sem=recv_sems.at[slot],
                device_id=(my_x, dst, my_z),
                device_id_type=pl.DeviceIdType.MESH,
            )
            rdma.start()
            sends.append(rdma)

        for slot in range(N_Y - 1):
            recv = pltpu.make_async_remote_copy(
                src_ref=recv_buf.at[slot],
                dst_ref=recv_buf.at[slot],
                send_sem=send_sems.at[slot],
                recv_sem=recv_sems.at[slot],
                device_id=(my_x, my_y, my_z),
                device_id_type=pl.DeviceIdType.MESH,
            )
            recv.wait_recv()

        out_ref[pl.ds(row0, m_half), :] = (
            x_ref[pl.ds(row0, m_half), pl.ds(my_y * n_chunk, n_chunk)]
            + recv_buf[0, :, :]
            + recv_buf[1, :, :]
            + recv_buf[2, :, :]
        )

        xchg = pltpu.make_async_remote_copy(
            src_ref=out_ref.at[pl.ds(row0, m_half), :],
            dst_ref=out_ref.at[pl.ds(row0, m_half), :],
            send_sem=send_sem_x,
            recv_sem=recv_sem_x,
            device_id=(partner_x, my_y, my_z),
            device_id_type=pl.DeviceIdType.MESH,
        )
        xchg.start()
        xrecv = pltpu.make_async_remote_copy(
            src_ref=out_ref.at[pl.ds(prow0, m_half), :],
            dst_ref=out_ref.at[pl.ds(prow0, m_half), :],
            send_sem=send_sem_x,
            recv_sem=recv_sem_x,
            device_id=(partner_x, my_y, my_z),
            device_id_type=pl.DeviceIdType.MESH,
        )
        xrecv.wait_recv()
        xchg.wait_send()

        for rdma in sends:
            rdma.wait_send()

    return pl.pallas_call(
        body,
        out_shape=jax.ShapeDtypeStruct((m, n_chunk), x2.dtype),
        in_specs=[pl.BlockSpec(memory_space=pltpu.VMEM)],
        out_specs=pl.BlockSpec(memory_space=pltpu.VMEM),
        scratch_shapes=[
            pltpu.VMEM((N_Y - 1, m_half, n_chunk), x2.dtype),
            pltpu.SemaphoreType.DMA((N_Y - 1,)),
            pltpu.SemaphoreType.DMA((N_Y - 1,)),
            pltpu.SemaphoreType.DMA,
            pltpu.SemaphoreType.DMA,
        ],
        compiler_params=pltpu.CompilerParams(collective_id=0),
    )(x2)
